# baseline (speedup 1.0000x reference)
import functools

import jax
import jax.numpy as jnp
from jax import lax
from jax.experimental import pallas as pl
from jax.experimental.pallas import tpu as pltpu
from jax.experimental.pallas import tpu_sc as plsc

D_MODEL = 64
SCALE = 8.0
B0 = 4096
B1 = 200
B_TOTAL = B0 * B1
N_PAIR = 500000
NUM_WORKERS = 32
PER_W = B_TOTAL // NUM_WORKERS
CHUNK = 256
N_CHUNKS = PER_W // CHUNK
IDX_W = 128
GPC = CHUNK // IDX_W
IDX_ROWS_PER_W = PER_W // IDX_W

_mesh = plsc.VectorSubcoreMesh(core_axis_name="c", subcore_axis_name="s")


@functools.partial(
    pl.kernel,
    mesh=_mesh,
    out_type=jax.ShapeDtypeStruct((B_TOTAL, 2 * D_MODEL), jnp.float32),
    compiler_params=pltpu.CompilerParams(needs_layout_passes=False),
    scratch_types=[
        pltpu.VMEM((GPC, IDX_W), jnp.int32),
        pltpu.VMEM((GPC, IDX_W), jnp.int32),
        pltpu.VMEM((GPC, IDX_W), jnp.int32),
        pltpu.VMEM((GPC, IDX_W), jnp.int32),
        pltpu.VMEM((CHUNK, 2 * D_MODEL), jnp.float32),
        pltpu.VMEM((CHUNK, 2 * D_MODEL), jnp.float32),
        pltpu.SemaphoreType.DMA,
        pltpu.SemaphoreType.DMA,
        pltpu.SemaphoreType.DMA,
        pltpu.SemaphoreType.DMA,
    ],
)
def _gather(idx_hbm, pair_hbm, tmp_hbm, ib0, ib1, sb0, sb1, rb0, rb1,
            gs0, gs1, os0, os1):
    wid = lax.axis_index("s") * 2 + lax.axis_index("c")
    ibufs = (ib0, ib1)
    sbufs = (sb0, sb1)
    rbufs = (rb0, rb1)
    gsems = (gs0, gs1)
    osems = (os0, os1)

    def load_idx(g, b):
        row = wid * IDX_ROWS_PER_W + g * GPC
        pltpu.sync_copy(idx_hbm.at[pl.ds(row, GPC)], ibufs[b])
        for j in range(GPC):
            for k in range(IDX_W // 16):
                sbufs[b][j, pl.ds(16 * k, 16)] = (
                    ibufs[b][j, pl.ds(16 * k, 16)] >> 1
                )

    def fire_gathers(b):
        for j in range(GPC):
            pltpu.async_copy(
                pair_hbm.at[sbufs[b].at[j]],
                rbufs[b].at[pl.ds(j * IDX_W, IDX_W)],
                gsems[b],
            )

    def wait_gathers(b):
        for j in range(GPC):
            pltpu.make_async_copy(
                pair_hbm.at[sbufs[b].at[j]],
                rbufs[b].at[pl.ds(j * IDX_W, IDX_W)],
                gsems[b],
            ).wait()

    def fire_out(g, b):
        base = wid * PER_W + g * CHUNK
        pltpu.async_copy(rbufs[b], tmp_hbm.at[pl.ds(base, CHUNK)], osems[b])

    def wait_out(g, b):
        base = wid * PER_W + g * CHUNK
        pltpu.make_async_copy(
            rbufs[b], tmp_hbm.at[pl.ds(base, CHUNK)], osems[b]
        ).wait()

    load_idx(0, 0)
    fire_gathers(0)
    for g in range(N_CHUNKS):
        b = g & 1
        nb = 1 - b
        if g + 1 < N_CHUNKS:
            load_idx(g + 1, nb)
            if g >= 1:
                wait_out(g - 1, nb)
            fire_gathers(nb)
        wait_gathers(b)
        fire_out(g, b)
    wait_out(N_CHUNKS - 2, (N_CHUNKS - 2) & 1)
    wait_out(N_CHUNKS - 1, (N_CHUNKS - 1) & 1)


_SUB = 512
_N_SUB = B0 // _SUB


def _xpose_body(t_ref, p_ref, o_ref):
    blk = t_ref[...]
    par = p_ref[0]
    pieces = []
    for m in range(_N_SUB):
        sub = blk[m * _SUB:(m + 1) * _SUB]
        pm = (par[m] == 1)[None, :]
        pieces.append(
            jnp.where(pm, sub[:, D_MODEL:].T, sub[:, :D_MODEL].T)
        )
    o_ref[...] = (jnp.concatenate(pieces, axis=1) * SCALE)[None]


@jax.jit
def _xpose(tmp, xpar):
    return pl.pallas_call(
        _xpose_body,
        grid=(B1,),
        in_specs=[
            pl.BlockSpec((B0, 2 * D_MODEL), lambda i: (i, 0)),
            pl.BlockSpec((1, _N_SUB, _SUB), lambda i: (i, 0, 0)),
        ],
        out_specs=pl.BlockSpec((1, D_MODEL, B0), lambda i: (i, 0, 0)),
        out_shape=jax.ShapeDtypeStruct((B1, D_MODEL, B0), jnp.float32),
    )(tmp, xpar)


def kernel(x, lut):
    xt = x.T.astype(jnp.int32)
    xf = xt.reshape(B_TOTAL // IDX_W, IDX_W)
    pairs = lut.reshape(N_PAIR, 2 * D_MODEL)
    tmp = _gather(xf, pairs)
    xpar = (xt & 1).reshape(B1, _N_SUB, _SUB)
    out_t = _xpose(tmp, xpar)
    return jnp.transpose(out_t, (2, 0, 1))
